# baseline (device time: 333276 ns/iter reference)
import jax
import jax.numpy as jnp
from jax import lax
from jax.experimental import pallas as pl
from jax.experimental.pallas import tpu as pltpu

N_DEV = 4
M_LOC = 1024
K_LOC = 1024
N_GLB = 8192
N_B = 512
N_BLOCKS = N_GLB // N_B
N_BLK_DIR = N_BLOCKS // 2
N_HOPS = N_DEV - 1

BF16 = jnp.bfloat16
F32 = jnp.float32


def kernel(x, w_mat):
    def body(x_hbm, w_hbm, z_hbm, out_hbm, x_bf, x_stage, w_bf, w_stage,
             comm_r, comm_l, amax_ref,
             send_r, recv_r, send_l, recv_l, *rest):
        cred_sems = {}
        idx = 0
        for _dirn in (0, 1):
            for _h in range(N_HOPS):
                for _par in (0, 1):
                    cred_sems[_dirn, _h, _par] = rest[idx]
                    idx += 1
        (x_sem, w_sems, o_sems, qr_sems, qw_sems,
         a_send_sems, a_recv_sems) = rest[12:]

        d = lax.axis_index("i")
        left = lax.rem(d + N_DEV - 1, N_DEV)
        right = lax.rem(d + 1, N_DEV)

        def conv_x(c):
            rows = pl.ds(c * M_LOC, M_LOC)
            dma = pltpu.make_async_copy(x_hbm.at[rows, :], x_stage, x_sem)
            dma.start()
            dma.wait()
            x_bf[rows, :] = x_stage[...].astype(BF16)

        barrier_sem = pltpu.get_barrier_semaphore()
        for nbr in (left, right):
            pl.semaphore_signal(
                barrier_sem, inc=1,
                device_id=(nbr,), device_id_type=pl.DeviceIdType.MESH,
            )
        pl.semaphore_wait(barrier_sem, 2)

        comm = (comm_r, comm_l)
        ssem = (send_r, send_l)
        rsem = (recv_r, recv_l)
        down = (right, left)
        up = (left, right)

        cols = lambda b: pl.ds(b * N_B, N_B)

        def w_dma(g, slot):
            return pltpu.make_async_copy(
                w_hbm.at[:, cols(g)], w_stage.at[slot], w_sems.at[slot])

        w_state = {"next": 0}
        w_dma(0, 0).start()
        w_dma(1, 1).start()

        def convert_w(n):
            for _ in range(n):
                g = w_state["next"]
                if g >= N_BLOCKS:
                    return
                slot = g % 2
                w_dma(g, slot).wait()
                w_bf[:, cols(g)] = w_stage[slot].astype(BF16)
                if g + 2 < N_BLOCKS:
                    w_dma(g + 2, slot).start()
                w_state["next"] = g + 1

        convert_w(2)
        conv_x(lax.rem(d + 3, N_DEV))
        conv_x(lax.rem(d + 1, N_DEV))

        def partial(chunk, b):
            xa = x_bf[pl.ds(chunk * M_LOC, M_LOC), :]
            wb = w_bf[:, cols(b)]
            return jnp.dot(xa, wb, preferred_element_type=F32)

        def chunk_of(dirn, stage):
            if stage == 3:
                return d
            off = (3 - stage) if dirn == 0 else (1 + stage)
            return lax.rem(d + off, N_DEV)

        def make_rdma(dirn, h, par):
            return pltpu.make_async_remote_copy(
                src_ref=comm[dirn].at[par, h],
                dst_ref=comm[dirn].at[par, h + 1],
                send_sem=ssem[dirn].at[h, par],
                recv_sem=rsem[dirn].at[h, par],
                device_id=(down[dirn],),
                device_id_type=pl.DeviceIdType.MESH,
            )

        def send_credit(dirn, h, par):
            pl.semaphore_signal(
                cred_sems[dirn, h, par], inc=1,
                device_id=(up[dirn],), device_id_type=pl.DeviceIdType.MESH,
            )

        rd = {}
        odesc = {}
        ocount = {"n": 0}
        amax_local = [jnp.float32(0.0)]

        def op(dirn, stage, j):
            bg = 2 * j + dirn
            par = j & 1
            p = partial(chunk_of(dirn, stage), bg)
            if stage == 0:
                if j >= 2:
                    rd[dirn, 1, j - 2].wait_send()
                    send_credit(dirn, 0, par)
                    rd[dirn, 0, j - 2].wait_send()
                comm[dirn][par, 0] = p.astype(BF16)
                if j >= 2:
                    pl.semaphore_wait(cred_sems[dirn, 0, par], 1)
                r = make_rdma(dirn, 0, par)
                r.start()
                rd[dirn, 0, j] = r
            elif stage in (1, 2):
                h = stage - 1
                if stage == 1 and j >= 2:
                    rd[dirn, 2, j - 2].wait_send()
                    send_credit(dirn, 1, par)
                rd[dirn, h, j].wait_recv()
                acc = comm[dirn][par, h + 1].astype(F32) + p
                comm[dirn][par, h + 1] = acc.astype(BF16)
                if j >= 2:
                    pl.semaphore_wait(cred_sems[dirn, stage, par], 1)
                r = make_rdma(dirn, h + 1, par)
                r.start()
                rd[dirn, h + 1, j] = r
            else:
                rd[dirn, 2, j].wait_recv()
                y = jnp.maximum(comm[dirn][par, N_HOPS].astype(F32) + p, 0.0)
                amax_local[0] = jnp.maximum(amax_local[0], jnp.max(y))
                i = ocount["n"]
                ocount["n"] = i + 1
                oslot = i % 2
                if i >= 2:
                    odesc[oslot].wait()
                o_stage[oslot] = y.astype(BF16)
                dma = pltpu.make_async_copy(
                    o_stage.at[oslot], out_hbm.at[:, cols(bg)],
                    o_sems.at[oslot])
                dma.start()
                odesc[oslot] = dma
                if j + 2 < N_BLK_DIR:
                    send_credit(dirn, 2, par)

        for r in range(N_BLK_DIR + 3):
            convert_w(2)
            for stage in range(4):
                j = r - stage
                if 0 <= j < N_BLK_DIR:
                    op(0, stage, j)
                    op(1, stage, j)
            if r == 0:
                conv_x(lax.rem(d + 2, N_DEV))
            elif r == 1:
                conv_x(d)

        for dirn in (0, 1):
            for h in range(N_HOPS):
                for j in (N_BLK_DIR - 2, N_BLK_DIR - 1):
                    rd[dirn, h, j].wait_send()
        odesc[0].wait()
        odesc[1].wait()

        amax_ref[pl.ds(d, 1)] = jnp.full((1, 8, 128), amax_local[0], F32)
        my_slot_src = amax_ref.at[pl.ds(d, 1)]
        for off in range(1, N_DEV):
            o = lax.rem(d + off, N_DEV)
            r = pltpu.make_async_remote_copy(
                src_ref=my_slot_src,
                dst_ref=my_slot_src,
                send_sem=a_send_sems.at[off - 1],
                recv_sem=a_recv_sems.at[off - 1],
                device_id=(o,),
                device_id_type=pl.DeviceIdType.MESH,
            )
            r.start()
            r.wait_send()
        for off in range(1, N_DEV):
            s = lax.rem(d + N_DEV - off, N_DEV)
            recv = pltpu.make_async_remote_copy(
                src_ref=my_slot_src,
                dst_ref=amax_ref.at[pl.ds(s, 1)],
                send_sem=a_send_sems.at[off - 1],
                recv_sem=a_recv_sems.at[off - 1],
                device_id=(s,),
                device_id_type=pl.DeviceIdType.MESH,
            )
            recv.wait_recv()
        g_amax = jnp.max(amax_ref[...])

        scale = g_amax / 127.0
        inv = 127.0 / g_amax

        def qread(g, slot):
            return pltpu.make_async_copy(
                out_hbm.at[:, cols(g)], q_in.at[slot], qr_sems.at[slot])

        qr = {0: qread(0, 0), 1: qread(1, 1)}
        qr[0].start()
        qr[1].start()
        qw = {}
        for g in range(N_BLOCKS):
            slot = g % 2
            qr[g].wait()
            y = q_in[slot].astype(F32)
            q = jnp.clip(jnp.round(y * inv), 0.0, 127.0)
            if g >= 2:
                qw[g - 2].wait()
            q_out[slot] = (q * scale).astype(BF16)
            if g + 2 < N_BLOCKS:
                qr[g + 2] = qread(g + 2, slot)
                qr[g + 2].start()
            dma = pltpu.make_async_copy(
                q_out.at[slot], out_hbm.at[:, cols(g)], qw_sems.at[slot])
            dma.start()
            qw[g] = dma
        qw[N_BLOCKS - 2].wait()
        qw[N_BLOCKS - 1].wait()

    o_stage = None
    q_in = None
    q_out = None

    def body_wrapper(x_hbm, w_hbm, z_hbm, out_hbm, x_bf, x_stage, w_bf,
                     w_stage, o_st, qi, qo, comm_r, comm_l, amax_ref,
                     send_r, recv_r, send_l, recv_l, *rest):
        nonlocal o_stage, q_in, q_out
        o_stage, q_in, q_out = o_st, qi, qo
        body(x_hbm, w_hbm, z_hbm, out_hbm, x_bf, x_stage, w_bf, w_stage,
             comm_r, comm_l, amax_ref,
             send_r, recv_r, send_l, recv_l, *rest)

    return pl.pallas_call(
        body_wrapper,
        out_shape=jax.ShapeDtypeStruct((M_LOC, N_GLB), BF16),
        in_specs=[
            pl.BlockSpec(memory_space=pl.ANY),
            pl.BlockSpec(memory_space=pl.ANY),
            pl.BlockSpec(memory_space=pl.ANY),
        ],
        out_specs=pl.BlockSpec(memory_space=pl.ANY),
        input_output_aliases={2: 0},
        scratch_shapes=[
            pltpu.VMEM((N_DEV * M_LOC, K_LOC), BF16),
            pltpu.VMEM((M_LOC, K_LOC), F32),
            pltpu.VMEM((K_LOC, N_GLB), BF16),
            pltpu.VMEM((2, K_LOC, N_B), F32),
            pltpu.VMEM((2, M_LOC, N_B), BF16),
            pltpu.VMEM((2, M_LOC, N_B), BF16),
            pltpu.VMEM((2, M_LOC, N_B), BF16),
            pltpu.VMEM((2, N_HOPS + 1, M_LOC, N_B), BF16),
            pltpu.VMEM((2, N_HOPS + 1, M_LOC, N_B), BF16),
            pltpu.VMEM((N_DEV, 8, 128), F32),
            pltpu.SemaphoreType.DMA((N_HOPS, 2)),
            pltpu.SemaphoreType.DMA((N_HOPS, 2)),
            pltpu.SemaphoreType.DMA((N_HOPS, 2)),
            pltpu.SemaphoreType.DMA((N_HOPS, 2)),
        ]
        + [pltpu.SemaphoreType.REGULAR] * 12
        + [
            pltpu.SemaphoreType.DMA,
            pltpu.SemaphoreType.DMA((2,)),
            pltpu.SemaphoreType.DMA((2,)),
            pltpu.SemaphoreType.DMA((2,)),
            pltpu.SemaphoreType.DMA((2,)),
            pltpu.SemaphoreType.DMA((N_DEV - 1,)),
            pltpu.SemaphoreType.DMA((N_DEV - 1,)),
        ],
        compiler_params=pltpu.CompilerParams(
            collective_id=0,
            vmem_limit_bytes=64 * 1024 * 1024,
        ),
    )(x, w_mat, jnp.zeros((M_LOC, N_GLB), BF16))


# device time: 326528 ns/iter; 1.0207x vs baseline; 1.0207x over previous
import jax
import jax.numpy as jnp
from jax import lax
from jax.experimental import pallas as pl
from jax.experimental.pallas import tpu as pltpu

N_DEV = 4
M_LOC = 1024
K_LOC = 1024
N_GLB = 8192
N_B = 512
N_BLOCKS = N_GLB // N_B
N_BLK_DIR = N_BLOCKS // 2
N_HOPS = N_DEV - 1

BF16 = jnp.bfloat16
F32 = jnp.float32


def kernel(x, w_mat):
    def body(x_hbm, w_hbm, out_hbm, x_bf, x_stage, w_bf, w_stage,
             comm_r, comm_l, amax_ref,
             send_r, recv_r, send_l, recv_l, *rest):
        cred_sems = {}
        idx = 0
        for _dirn in (0, 1):
            for _h in range(N_HOPS):
                for _par in (0, 1):
                    cred_sems[_dirn, _h, _par] = rest[idx]
                    idx += 1
        (x_sem, w_sems, o_sems, qr_sems, qw_sems,
         a_send_sems, a_recv_sems) = rest[12:]

        d = lax.axis_index("i")
        left = lax.rem(d + N_DEV - 1, N_DEV)
        right = lax.rem(d + 1, N_DEV)

        def conv_x(c):
            rows = pl.ds(c * M_LOC, M_LOC)
            dma = pltpu.make_async_copy(x_hbm.at[rows, :], x_stage, x_sem)
            dma.start()
            dma.wait()
            x_bf[rows, :] = x_stage[...].astype(BF16)

        barrier_sem = pltpu.get_barrier_semaphore()
        for nbr in (left, right):
            pl.semaphore_signal(
                barrier_sem, inc=1,
                device_id=(nbr,), device_id_type=pl.DeviceIdType.MESH,
            )
        pl.semaphore_wait(barrier_sem, 2)

        comm = (comm_r, comm_l)
        ssem = (send_r, send_l)
        rsem = (recv_r, recv_l)
        down = (right, left)
        up = (left, right)

        cols = lambda b: pl.ds(b * N_B, N_B)

        def w_dma(g, slot):
            return pltpu.make_async_copy(
                w_hbm.at[:, cols(g)], w_stage.at[slot], w_sems.at[slot])

        w_state = {"next": 0}
        w_dma(0, 0).start()
        w_dma(1, 1).start()

        def convert_w(n):
            for _ in range(n):
                g = w_state["next"]
                if g >= N_BLOCKS:
                    return
                slot = g % 2
                w_dma(g, slot).wait()
                w_bf[:, cols(g)] = w_stage[slot].astype(BF16)
                if g + 2 < N_BLOCKS:
                    w_dma(g + 2, slot).start()
                w_state["next"] = g + 1

        convert_w(2)
        conv_x(lax.rem(d + 3, N_DEV))
        conv_x(lax.rem(d + 1, N_DEV))

        def partial(chunk, b):
            xa = x_bf[pl.ds(chunk * M_LOC, M_LOC), :]
            wb = w_bf[:, cols(b)]
            return jnp.dot(xa, wb, preferred_element_type=F32)

        def chunk_of(dirn, stage):
            if stage == 3:
                return d
            off = (3 - stage) if dirn == 0 else (1 + stage)
            return lax.rem(d + off, N_DEV)

        def make_rdma(dirn, h, par):
            return pltpu.make_async_remote_copy(
                src_ref=comm[dirn].at[par, h],
                dst_ref=comm[dirn].at[par, h + 1],
                send_sem=ssem[dirn].at[h, par],
                recv_sem=rsem[dirn].at[h, par],
                device_id=(down[dirn],),
                device_id_type=pl.DeviceIdType.MESH,
            )

        def send_credit(dirn, h, par):
            pl.semaphore_signal(
                cred_sems[dirn, h, par], inc=1,
                device_id=(up[dirn],), device_id_type=pl.DeviceIdType.MESH,
            )

        rd = {}
        odesc = {}
        ocount = {"n": 0}
        amax_local = [jnp.float32(0.0)]

        def op(dirn, stage, j):
            bg = 2 * j + dirn
            par = j & 1
            p = partial(chunk_of(dirn, stage), bg)
            if stage == 0:
                if j >= 2:
                    rd[dirn, 1, j - 2].wait_send()
                    send_credit(dirn, 0, par)
                    rd[dirn, 0, j - 2].wait_send()
                comm[dirn][par, 0] = p.astype(BF16)
                if j >= 2:
                    pl.semaphore_wait(cred_sems[dirn, 0, par], 1)
                r = make_rdma(dirn, 0, par)
                r.start()
                rd[dirn, 0, j] = r
            elif stage in (1, 2):
                h = stage - 1
                if stage == 1 and j >= 2:
                    rd[dirn, 2, j - 2].wait_send()
                    send_credit(dirn, 1, par)
                rd[dirn, h, j].wait_recv()
                acc = comm[dirn][par, h + 1].astype(F32) + p
                comm[dirn][par, h + 1] = acc.astype(BF16)
                if j >= 2:
                    pl.semaphore_wait(cred_sems[dirn, stage, par], 1)
                r = make_rdma(dirn, h + 1, par)
                r.start()
                rd[dirn, h + 1, j] = r
            else:
                rd[dirn, 2, j].wait_recv()
                y = jnp.maximum(comm[dirn][par, N_HOPS].astype(F32) + p, 0.0)
                amax_local[0] = jnp.maximum(amax_local[0], jnp.max(y))
                i = ocount["n"]
                ocount["n"] = i + 1
                oslot = i % 2
                if i >= 2:
                    odesc[oslot].wait()
                o_stage[oslot] = y.astype(BF16)
                dma = pltpu.make_async_copy(
                    o_stage.at[oslot], out_hbm.at[:, cols(bg)],
                    o_sems.at[oslot])
                dma.start()
                odesc[oslot] = dma
                if j + 2 < N_BLK_DIR:
                    send_credit(dirn, 2, par)

        for r in range(N_BLK_DIR + 3):
            convert_w(2)
            for stage in range(4):
                j = r - stage
                if 0 <= j < N_BLK_DIR:
                    op(0, stage, j)
                    op(1, stage, j)
            if r == 0:
                conv_x(lax.rem(d + 2, N_DEV))
            elif r == 1:
                conv_x(d)

        for dirn in (0, 1):
            for h in range(N_HOPS):
                for j in (N_BLK_DIR - 2, N_BLK_DIR - 1):
                    rd[dirn, h, j].wait_send()
        odesc[0].wait()
        odesc[1].wait()

        amax_ref[pl.ds(d, 1)] = jnp.full((1, 8, 128), amax_local[0], F32)
        my_slot_src = amax_ref.at[pl.ds(d, 1)]
        for off in range(1, N_DEV):
            o = lax.rem(d + off, N_DEV)
            r = pltpu.make_async_remote_copy(
                src_ref=my_slot_src,
                dst_ref=my_slot_src,
                send_sem=a_send_sems.at[off - 1],
                recv_sem=a_recv_sems.at[off - 1],
                device_id=(o,),
                device_id_type=pl.DeviceIdType.MESH,
            )
            r.start()
            r.wait_send()
        for off in range(1, N_DEV):
            s = lax.rem(d + N_DEV - off, N_DEV)
            recv = pltpu.make_async_remote_copy(
                src_ref=my_slot_src,
                dst_ref=amax_ref.at[pl.ds(s, 1)],
                send_sem=a_send_sems.at[off - 1],
                recv_sem=a_recv_sems.at[off - 1],
                device_id=(s,),
                device_id_type=pl.DeviceIdType.MESH,
            )
            recv.wait_recv()
        g_amax = jnp.max(amax_ref[...])

        scale = g_amax / 127.0
        inv = 127.0 / g_amax

        def qread(g, slot):
            return pltpu.make_async_copy(
                out_hbm.at[:, cols(g)], q_in.at[slot], qr_sems.at[slot])

        qr = {0: qread(0, 0), 1: qread(1, 1)}
        qr[0].start()
        qr[1].start()
        qw = {}
        for g in range(N_BLOCKS):
            slot = g % 2
            qr[g].wait()
            y = q_in[slot].astype(F32)
            q = jnp.clip(jnp.round(y * inv), 0.0, 127.0)
            if g >= 2:
                qw[g - 2].wait()
            q_out[slot] = (q * scale).astype(BF16)
            if g + 2 < N_BLOCKS:
                qr[g + 2] = qread(g + 2, slot)
                qr[g + 2].start()
            dma = pltpu.make_async_copy(
                q_out.at[slot], out_hbm.at[:, cols(g)], qw_sems.at[slot])
            dma.start()
            qw[g] = dma
        qw[N_BLOCKS - 2].wait()
        qw[N_BLOCKS - 1].wait()

    o_stage = None
    q_in = None
    q_out = None

    def body_wrapper(x_hbm, w_hbm, out_hbm, x_bf, x_stage, w_bf, w_stage,
                     o_st, qi, qo, comm_r, comm_l, amax_ref,
                     send_r, recv_r, send_l, recv_l, *rest):
        nonlocal o_stage, q_in, q_out
        o_stage, q_in, q_out = o_st, qi, qo
        body(x_hbm, w_hbm, out_hbm, x_bf, x_stage, w_bf, w_stage,
             comm_r, comm_l, amax_ref,
             send_r, recv_r, send_l, recv_l, *rest)

    return pl.pallas_call(
        body_wrapper,
        out_shape=jax.ShapeDtypeStruct((M_LOC, N_GLB), BF16),
        in_specs=[
            pl.BlockSpec(memory_space=pl.ANY),
            pl.BlockSpec(memory_space=pl.ANY),
        ],
        out_specs=pl.BlockSpec(memory_space=pl.ANY),
        scratch_shapes=[
            pltpu.VMEM((N_DEV * M_LOC, K_LOC), BF16),
            pltpu.VMEM((M_LOC, K_LOC), F32),
            pltpu.VMEM((K_LOC, N_GLB), BF16),
            pltpu.VMEM((2, K_LOC, N_B), F32),
            pltpu.VMEM((2, M_LOC, N_B), BF16),
            pltpu.VMEM((2, M_LOC, N_B), BF16),
            pltpu.VMEM((2, M_LOC, N_B), BF16),
            pltpu.VMEM((2, N_HOPS + 1, M_LOC, N_B), BF16),
            pltpu.VMEM((2, N_HOPS + 1, M_LOC, N_B), BF16),
            pltpu.VMEM((N_DEV, 8, 128), F32),
            pltpu.SemaphoreType.DMA((N_HOPS, 2)),
            pltpu.SemaphoreType.DMA((N_HOPS, 2)),
            pltpu.SemaphoreType.DMA((N_HOPS, 2)),
            pltpu.SemaphoreType.DMA((N_HOPS, 2)),
        ]
        + [pltpu.SemaphoreType.REGULAR] * 12
        + [
            pltpu.SemaphoreType.DMA,
            pltpu.SemaphoreType.DMA((2,)),
            pltpu.SemaphoreType.DMA((2,)),
            pltpu.SemaphoreType.DMA((2,)),
            pltpu.SemaphoreType.DMA((2,)),
            pltpu.SemaphoreType.DMA((N_DEV - 1,)),
            pltpu.SemaphoreType.DMA((N_DEV - 1,)),
        ],
        compiler_params=pltpu.CompilerParams(
            collective_id=0,
            vmem_limit_bytes=64 * 1024 * 1024,
        ),
    )(x, w_mat)


kernel.__doc__ = __doc__
